# baseline (speedup 1.0000x reference)
import functools
import math

import jax
import jax.numpy as jnp
from jax import lax
from jax.experimental import pallas as pl
from jax.experimental.pallas import tpu as pltpu
from jax.experimental.pallas import tpu_sc as plsc

_EMB = 128
_SCALE = math.sqrt(float(_EMB))
_NC = 2
_NS = 16
_NW = _NC * _NS
_K = 128
_LANES = 16
_P = 3
_D = 2 * _P


@functools.lru_cache(maxsize=None)
def _emb_call(nchunk):
    mesh = plsc.VectorSubcoreMesh(core_axis_name="c", subcore_axis_name="s")

    @functools.partial(
        pl.kernel,
        mesh=mesh,
        out_type=jax.ShapeDtypeStruct((_NW, nchunk, _K, _EMB), jnp.float32),
        scratch_types=[
            pltpu.VMEM((nchunk, _K), jnp.int32),
            pltpu.VMEM((_D, _K, _EMB), jnp.float32),
            pltpu.SemaphoreType.DMA,
            pltpu.SemaphoreType.DMA,
        ],
    )
    def body(tok_hbm, table_hbm, out_hbm, idx_v, rows_v, gsem, ssem):
        wid = lax.axis_index("s") * _NC + lax.axis_index("c")
        pltpu.sync_copy(tok_hbm.at[wid], idx_v)

        for jj in range(_P):
            pltpu.async_copy(table_hbm.at[idx_v.at[jj]], rows_v.at[jj], gsem)

        def chunk(j, carry):
            @pl.when(j >= _P)
            def _():
                pltpu.make_async_copy(rows_v.at[0], out_hbm.at[wid, 0],
                                      ssem).wait()

            @pl.when(j + _P < nchunk)
            def _():
                pslot = lax.rem(j + _P, _D)
                pltpu.async_copy(table_hbm.at[idx_v.at[j + _P]],
                                 rows_v.at[pslot], gsem)

            slot = lax.rem(j, _D)
            pltpu.make_async_copy(table_hbm.at[idx_v.at[j]], rows_v.at[slot],
                                  gsem).wait()

            def scale_rows(r2, c2):
                for u in range(2):
                    for c in range(_EMB // _LANES):
                        sl = (slot, r2 * 2 + u, pl.ds(c * _LANES, _LANES))
                        rows_v[sl] = rows_v[sl] * _SCALE
                return c2

            lax.fori_loop(0, _K // 2, scale_rows, 0)
            pltpu.async_copy(rows_v.at[slot], out_hbm.at[wid, j], ssem)
            return carry

        lax.fori_loop(0, nchunk, chunk, 0)

        for _jj in range(_P):
            pltpu.make_async_copy(rows_v.at[0], out_hbm.at[wid, 0], ssem).wait()

    return body


def kernel(tokens, table):
    b, s = tokens.shape
    rows = b * s
    assert rows % (_NW * _K) == 0
    nchunk = rows // (_NW * _K)
    tok = tokens.reshape(_NW, nchunk, _K).astype(jnp.int32)
    out = _emb_call(nchunk)(tok, table)
    return out.reshape(b, s, _EMB)


# speedup vs baseline: 1.6707x; 1.6707x over previous
import functools
import math

import jax
import jax.numpy as jnp
from jax import lax
from jax.experimental import pallas as pl
from jax.experimental.pallas import tpu as pltpu
from jax.experimental.pallas import tpu_sc as plsc

_EMB = 128
_SCALE = math.sqrt(float(_EMB))
_NC = 2
_NS = 16
_NW = _NC * _NS
_K = 128
_LANES = 16
_P = 2
_D = 5


@functools.lru_cache(maxsize=None)
def _emb_call(nchunk):
    assert nchunk % _D == 0
    nretire = _D - _P
    mesh = plsc.VectorSubcoreMesh(core_axis_name="c", subcore_axis_name="s")

    @functools.partial(
        pl.kernel,
        mesh=mesh,
        out_type=jax.ShapeDtypeStruct((_NW, nchunk, _K, _EMB), jnp.float32),
        scratch_types=[
            pltpu.VMEM((nchunk, _K), jnp.int32),
            pltpu.VMEM((_D, _K, _EMB), jnp.float32),
            pltpu.SemaphoreType.DMA,
            pltpu.SemaphoreType.DMA,
        ],
    )
    def body(tok_hbm, table_hbm, out_hbm, idx_v, rows_v, gsem, ssem):
        wid = lax.axis_index("s") * _NC + lax.axis_index("c")
        pltpu.sync_copy(tok_hbm.at[wid], idx_v)

        for jj in range(_P):
            pltpu.async_copy(table_hbm.at[idx_v.at[jj]], rows_v.at[jj], gsem)

        def group(o, carry):
            j0 = o * _D
            for b in range(_D):
                j = j0 + b
                @pl.when(j >= nretire)
                def _():
                    pltpu.make_async_copy(rows_v.at[0], out_hbm.at[wid, 0],
                                          ssem).wait()

                @pl.when(j + _P < nchunk)
                def _():
                    pltpu.async_copy(table_hbm.at[idx_v.at[j + _P]],
                                     rows_v.at[(b + _P) % _D], gsem)

                pltpu.make_async_copy(table_hbm.at[idx_v.at[j]], rows_v.at[b],
                                      gsem).wait()

                def scale_rows(r2, c2, b=b):
                    for u in range(2):
                        for c in range(_EMB // _LANES):
                            sl = (b, r2 * 2 + u, pl.ds(c * _LANES, _LANES))
                            rows_v[sl] = rows_v[sl] * _SCALE
                    return c2

                lax.fori_loop(0, _K // 2, scale_rows, 0)
                pltpu.async_copy(rows_v.at[b], out_hbm.at[wid, j], ssem)
            return carry

        lax.fori_loop(0, nchunk // _D, group, 0)

        for _jj in range(nretire):
            pltpu.make_async_copy(rows_v.at[0], out_hbm.at[wid, 0], ssem).wait()

    return body


def kernel(tokens, table):
    b, s = tokens.shape
    rows = b * s
    assert rows % (_NW * _K) == 0
    nchunk = rows // (_NW * _K)
    tok = tokens.reshape(_NW, nchunk, _K).astype(jnp.int32)
    out = _emb_call(nchunk)(tok, table)
    return out.reshape(b, s, _EMB)


# speedup vs baseline: 2.9695x; 1.7773x over previous
import functools
import math

import jax
import jax.numpy as jnp
from jax import lax
from jax.experimental import pallas as pl
from jax.experimental.pallas import tpu as pltpu
from jax.experimental.pallas import tpu_sc as plsc

_EMB = 128
_SCALE = math.sqrt(float(_EMB))
_NC = 2
_NS = 16
_NW = _NC * _NS
_BPC = 2
_LANES = 16
_P = 2
_D = 4


@functools.lru_cache(maxsize=None)
def _emb_call(batch, seq):
    rows_per_chunk = _BPC * seq
    batches_per_worker = batch // _NW
    nchunk = batches_per_worker // _BPC
    assert nchunk % _D == 0
    nretire = _D - _P
    mesh = plsc.VectorSubcoreMesh(core_axis_name="c", subcore_axis_name="s")

    @functools.partial(
        pl.kernel,
        mesh=mesh,
        out_type=jax.ShapeDtypeStruct((batch, seq, _EMB), jnp.float32),
        scratch_types=[
            pltpu.VMEM((nchunk, rows_per_chunk), jnp.int32),
            pltpu.VMEM((_D, rows_per_chunk, _EMB), jnp.float32),
            pltpu.SemaphoreType.DMA,
            pltpu.SemaphoreType.DMA,
        ],
    )
    def body(tok_hbm, table_hbm, out_hbm, idx_v, rows_v, gsem, ssem):
        wid = lax.axis_index("s") * _NC + lax.axis_index("c")
        b0 = wid * batches_per_worker
        pltpu.sync_copy(tok_hbm.at[wid], idx_v)

        for jj in range(_P):
            pltpu.async_copy(table_hbm.at[idx_v.at[jj]], rows_v.at[jj], gsem)

        def group(o, carry):
            j0 = o * _D
            for b in range(_D):
                j = j0 + b
                @pl.when(j >= nretire)
                def _():
                    for u in range(_BPC):
                        pltpu.make_async_copy(
                            rows_v.at[0, pl.ds(u * seq, seq)],
                            out_hbm.at[0], ssem).wait()

                @pl.when(j + _P < nchunk)
                def _():
                    pltpu.async_copy(table_hbm.at[idx_v.at[j + _P]],
                                     rows_v.at[(b + _P) % _D], gsem)

                pltpu.make_async_copy(table_hbm.at[idx_v.at[j]], rows_v.at[b],
                                      gsem).wait()

                def scale_rows(r2, c2, b=b):
                    for u in range(2):
                        for c in range(_EMB // _LANES):
                            sl = (b, r2 * 2 + u, pl.ds(c * _LANES, _LANES))
                            rows_v[sl] = rows_v[sl] * _SCALE
                    return c2

                lax.fori_loop(0, rows_per_chunk // 2, scale_rows, 0)
                for u in range(_BPC):
                    pltpu.async_copy(rows_v.at[b, pl.ds(u * seq, seq)],
                                     out_hbm.at[b0 + j * _BPC + u], ssem)
            return carry

        lax.fori_loop(0, nchunk // _D, group, 0)

        for _jj in range(nretire * _BPC):
            pltpu.make_async_copy(rows_v.at[0, pl.ds(0, seq)],
                                  out_hbm.at[0], ssem).wait()

    return body


def kernel(tokens, table):
    batch, seq = tokens.shape
    assert batch % (_NW * _BPC) == 0
    nchunk = batch // (_NW * _BPC)
    tok = tokens.reshape(_NW, nchunk, _BPC * seq).astype(jnp.int32)
    return _emb_call(batch, seq)(tok, table)
